# baseline (speedup 1.0000x reference)
import jax
import jax.numpy as jnp
import numpy as np
from jax.experimental import pallas as pl

_B = 64
_M = 128
_F = 48
_C = 7
_H = 20
_N = _B * _M

_GP = 4

_SQRT2 = 1.4142135623730951

_G_BLOCK = np.kron(np.eye(_C, dtype=np.float32),
                   np.ones((_C, _C), dtype=np.float32))


def _gelu(x):
    return 0.5 * x * (1.0 + jax.lax.erf(x / _SQRT2))


def _elu(x):
    return jnp.maximum(x, 0.0) + (jnp.exp(jnp.minimum(x, 0.0)) - 1.0)


def _dinv(A, ones_col):
    deg = jax.lax.dot_general(A, ones_col, (((0,), (0,)), ((), ())),
                              preferred_element_type=jnp.float32)
    safe = jnp.where(deg > 0, deg, 1.0)
    return jnp.where(deg > 0, jax.lax.rsqrt(safe), 0.0)


def _dot(a, b):
    return jnp.dot(a, b, preferred_element_type=jnp.float32)


def _aggT(A, d, h):
    return d * jax.lax.dot_general(
        A, d * h, (((0,), (0,)), ((), ())),
        preferred_element_type=jnp.float32)


def _fused_kernel(Aa_ref, Aw_ref, Ae_ref, Al_ref,
                  xT_ref, xTw_ref, xTe_ref, xD_ref, xDu_ref, xL_ref,
                  WV1_ref, b1a_ref, b1s_ref, M2a_ref, M2s_ref,
                  b2a_ref, b2s_ref,
                  L1_ref, bl1_ref, L2_ref, bl2_ref,
                  G_ref, P_ref, bf_ref,
                  out_ref):
    G = G_ref[...]
    ones_col = jnp.full((_M, 1), 1.0, dtype=jnp.float32)
    for i in range(_GP):
        Aa = Aa_ref[i]
        Aw = Aw_ref[i]
        Ae = Ae_ref[i]
        Al = Al_ref[i]
        da = _dinv(Aa, ones_col)
        dw = _dinv(Aw, ones_col)
        de = _dinv(Ae, ones_col)
        dl = _dinv(Al, ones_col)

        hT = _dot(xT_ref[i], WV1_ref[0])
        hD = _dot(xD_ref[i], WV1_ref[1])
        hDu = _dot(xDu_ref[i], WV1_ref[2])
        hW = _dot(xTw_ref[i], WV1_ref[3])
        hE = _dot(xTe_ref[i], WV1_ref[4])
        hL = _dot(xL_ref[i], WV1_ref[5])

        hw_a = jnp.concatenate([hT[:, :_H], hD[:, :_H], hDu[:, :_H]], axis=1)
        hv_a = jnp.concatenate([hT[:, _H:], hD[:, _H:], hDu[:, _H:]], axis=1)
        h1a = _elu(_gelu(_aggT(Aa, da, hw_a) + hv_a + b1a_ref[...]))
        h1w = _elu(_gelu(_aggT(Aw, dw, hW[:, :_H]) + hW[:, _H:] + b1s_ref[0]))
        h1e = _elu(_gelu(_aggT(Ae, de, hE[:, :_H]) + hE[:, _H:] + b1s_ref[1]))
        h1l = _elu(_gelu(_aggT(Al, dl, hL[:, :_H]) + hL[:, _H:] + b1s_ref[2]))

        H2a = _dot(h1a, M2a_ref[...])
        H2w = _dot(h1w, M2s_ref[0])
        H2e = _dot(h1e, M2s_ref[1])
        H2l = _dot(h1l, M2s_ref[2])
        s_a = jax.nn.relu(_aggT(Aa, da, H2a[:, :3 * _C])
                          + H2a[:, 3 * _C:] + b2a_ref[...])
        s_w = jax.nn.relu(_aggT(Aw, dw, H2w[:, :_C]) + H2w[:, _C:]
                          + b2s_ref[0])
        s_e = jax.nn.relu(_aggT(Ae, de, H2e[:, :_C]) + H2e[:, _C:]
                          + b2s_ref[1])
        s_l = jax.nn.relu(_aggT(Al, dl, H2l[:, :_C]) + H2l[:, _C:]
                          + b2s_ref[2])

        t = jax.nn.relu(_dot(xL_ref[i], L1_ref[...]) + bl1_ref[...])
        s_lt = _dot(t, L2_ref[...]) + bl2_ref[...]

        S = jnp.concatenate([s_a, s_w, s_e, s_l, s_lt], axis=1)
        E = jnp.exp(S - jnp.max(S, axis=-1, keepdims=True))
        En = E / _dot(E, G)
        out_ref[pl.ds(i * _M, _M), :] = _dot(En, P_ref[...]) + bf_ref[...]


def kernel(A_input, A_week_input, A_weekend_input, Location_location_input,
           Temporal_input, Temporal_week_input, Temporal_weekend_input,
           Distance_input, Duration_input, Location_time_input,
           W1_temporal, V1_temporal, b1_temporal, W2_temporal, V2_temporal, b2_temporal,
           W1_week, V1_week, b1_week, W2_week, V2_week, b2_week,
           W1_weekend, V1_weekend, b1_weekend, W2_weekend, V2_weekend, b2_weekend,
           W1_distance, V1_distance, b1_distance, W2_distance, V2_distance, b2_distance,
           W1_duration, V1_duration, b1_duration, W2_duration, V2_duration, b2_duration,
           W1_loctime, V1_loctime, b1_loctime, W2_loctime, V2_loctime, b2_loctime,
           L1, bl1, L2, bl2, Ld, bd, Lo, bo):
    WV1 = jnp.stack([
        jnp.concatenate([W1_temporal, V1_temporal], axis=1),
        jnp.concatenate([W1_distance, V1_distance], axis=1),
        jnp.concatenate([W1_duration, V1_duration], axis=1),
        jnp.concatenate([W1_week, V1_week], axis=1),
        jnp.concatenate([W1_weekend, V1_weekend], axis=1),
        jnp.concatenate([W1_loctime, V1_loctime], axis=1),
    ])
    b1a = jnp.concatenate([b1_temporal, b1_distance, b1_duration])[None, :]
    b1s = jnp.stack([b1_week, b1_weekend, b1_loctime])[:, None, :]

    z = jnp.zeros((_H, _C), jnp.float32)
    M2a = jnp.concatenate([
        jnp.concatenate([W2_temporal, z, z, V2_temporal, z, z], axis=1),
        jnp.concatenate([z, W2_distance, z, z, V2_distance, z], axis=1),
        jnp.concatenate([z, z, W2_duration, z, z, V2_duration], axis=1),
    ], axis=0)
    M2s = jnp.stack([
        jnp.concatenate([W2_week, V2_week], axis=1),
        jnp.concatenate([W2_weekend, V2_weekend], axis=1),
        jnp.concatenate([W2_loctime, V2_loctime], axis=1),
    ])
    b2a = jnp.concatenate([b2_temporal, b2_distance, b2_duration])[None, :]
    b2s = jnp.stack([b2_week, b2_weekend, b2_loctime])[:, None, :]

    P = jnp.concatenate([Lo, Lo, Lo, Lo, Lo, 2.0 * Ld, 2.0 * Ld], axis=0)
    bf = (bd + bo)[None, :]

    grid = (_B // _GP,)
    badj = pl.BlockSpec((_GP, _M, _M), lambda b: (b, 0, 0))
    bx = pl.BlockSpec((_GP, _M, _F), lambda b: (b, 0, 0))

    def bcast(shape):
        nd = len(shape)
        return pl.BlockSpec(shape, lambda b: (0,) * nd)

    out = pl.pallas_call(
        _fused_kernel,
        grid=grid,
        in_specs=[badj, badj, badj, badj,
                  bx, bx, bx, bx, bx, bx,
                  bcast((6, _F, 2 * _H)), bcast((1, 3 * _H)),
                  bcast((3, 1, _H)),
                  bcast((3 * _H, 6 * _C)), bcast((3, _H, 2 * _C)),
                  bcast((1, 3 * _C)), bcast((3, 1, _C)),
                  bcast((_F, 40)), bcast((1, 40)),
                  bcast((40, _C)), bcast((1, _C)),
                  bcast((_C * _C, _C * _C)),
                  bcast((_C * _C, _C)), bcast((1, _C))],
        out_specs=pl.BlockSpec((_GP * _M, _C), lambda b: (b, 0)),
        out_shape=jax.ShapeDtypeStruct((_N, _C), jnp.float32),
    )(A_input, A_week_input, A_weekend_input, Location_location_input,
      Temporal_input, Temporal_week_input, Temporal_weekend_input,
      Distance_input, Duration_input, Location_time_input,
      WV1, b1a, b1s, M2a, M2s, b2a, b2s,
      L1, bl1[None, :], L2, bl2[None, :], jnp.asarray(_G_BLOCK), P, bf)
    return out


# speedup vs baseline: 1.0709x; 1.0709x over previous
import jax
import jax.numpy as jnp
import numpy as np
from jax.experimental import pallas as pl

_B = 64
_M = 128
_F = 48
_C = 7
_H = 20
_N = _B * _M

_GP = 4

_SQRT2 = 1.4142135623730951

_G_BLOCK = np.kron(np.eye(_C, dtype=np.float32),
                   np.ones((_C, _C), dtype=np.float32))


def _gelu(x):
    return 0.5 * x * (1.0 + jax.lax.erf(x / _SQRT2))


def _elu(x):
    return jnp.maximum(x, 0.0) + (jnp.exp(jnp.minimum(x, 0.0)) - 1.0)


def _b16(x):
    return x.astype(jnp.bfloat16)


def _dinv(Ab, ones_col):
    deg = jax.lax.dot_general(Ab, ones_col, (((0,), (0,)), ((), ())),
                              preferred_element_type=jnp.float32)
    safe = jnp.where(deg > 0, deg, 1.0)
    return jnp.where(deg > 0, jax.lax.rsqrt(safe), 0.0)


def _dot(a, b):
    return jnp.dot(_b16(a), b, preferred_element_type=jnp.float32)


def _aggT(An_b, d, h):
    return d * jax.lax.dot_general(
        An_b, _b16(h), (((0,), (0,)), ((), ())),
        preferred_element_type=jnp.float32)


def _fused_kernel(Aa_ref, Aw_ref, Ae_ref, Al_ref,
                  xT_ref, xTw_ref, xTe_ref, xD_ref, xDu_ref, xL_ref,
                  WV1_ref, b1a_ref, b1s_ref, M2a_ref, M2s_ref,
                  b2a_ref, b2s_ref,
                  L1_ref, bl1_ref, L2_ref, bl2_ref,
                  G_ref, P_ref, bf_ref,
                  out_ref):
    G = G_ref[...]
    ones_col = jnp.full((_M, 1), 1.0, dtype=jnp.bfloat16)
    for i in range(_GP):
        Aa = Aa_ref[i]
        Aw = Aw_ref[i]
        Ae = Ae_ref[i]
        Al = Al_ref[i]
        da = _dinv(_b16(Aa), ones_col)
        dw = _dinv(_b16(Aw), ones_col)
        de = _dinv(_b16(Ae), ones_col)
        dl = _dinv(_b16(Al), ones_col)
        Aa = _b16(da * Aa)
        Aw = _b16(dw * Aw)
        Ae = _b16(de * Ae)
        Al = _b16(dl * Al)

        hT = _dot(xT_ref[i], WV1_ref[0])
        hD = _dot(xD_ref[i], WV1_ref[1])
        hDu = _dot(xDu_ref[i], WV1_ref[2])
        hW = _dot(xTw_ref[i], WV1_ref[3])
        hE = _dot(xTe_ref[i], WV1_ref[4])
        hL = _dot(xL_ref[i], WV1_ref[5])

        hw_a = jnp.concatenate([hT[:, :_H], hD[:, :_H], hDu[:, :_H]], axis=1)
        hv_a = jnp.concatenate([hT[:, _H:], hD[:, _H:], hDu[:, _H:]], axis=1)
        h1a = _elu(_gelu(_aggT(Aa, da, hw_a) + hv_a + b1a_ref[...]))
        h1w = _elu(_gelu(_aggT(Aw, dw, hW[:, :_H]) + hW[:, _H:] + b1s_ref[0]))
        h1e = _elu(_gelu(_aggT(Ae, de, hE[:, :_H]) + hE[:, _H:] + b1s_ref[1]))
        h1l = _elu(_gelu(_aggT(Al, dl, hL[:, :_H]) + hL[:, _H:] + b1s_ref[2]))

        H2a = _dot(h1a, M2a_ref[...])
        H2w = _dot(h1w, M2s_ref[0])
        H2e = _dot(h1e, M2s_ref[1])
        H2l = _dot(h1l, M2s_ref[2])
        s_a = jax.nn.relu(_aggT(Aa, da, H2a[:, :3 * _C])
                          + H2a[:, 3 * _C:] + b2a_ref[...])
        s_w = jax.nn.relu(_aggT(Aw, dw, H2w[:, :_C]) + H2w[:, _C:]
                          + b2s_ref[0])
        s_e = jax.nn.relu(_aggT(Ae, de, H2e[:, :_C]) + H2e[:, _C:]
                          + b2s_ref[1])
        s_l = jax.nn.relu(_aggT(Al, dl, H2l[:, :_C]) + H2l[:, _C:]
                          + b2s_ref[2])

        t = jax.nn.relu(_dot(xL_ref[i], L1_ref[...]) + bl1_ref[...])
        s_lt = _dot(t, L2_ref[...]) + bl2_ref[...]

        S = jnp.concatenate([s_a, s_w, s_e, s_l, s_lt], axis=1)
        E = jnp.exp(S - jnp.max(S, axis=-1, keepdims=True))
        En = E / _dot(E, G)
        out_ref[pl.ds(i * _M, _M), :] = _dot(En, P_ref[...]) + bf_ref[...]


def kernel(A_input, A_week_input, A_weekend_input, Location_location_input,
           Temporal_input, Temporal_week_input, Temporal_weekend_input,
           Distance_input, Duration_input, Location_time_input,
           W1_temporal, V1_temporal, b1_temporal, W2_temporal, V2_temporal, b2_temporal,
           W1_week, V1_week, b1_week, W2_week, V2_week, b2_week,
           W1_weekend, V1_weekend, b1_weekend, W2_weekend, V2_weekend, b2_weekend,
           W1_distance, V1_distance, b1_distance, W2_distance, V2_distance, b2_distance,
           W1_duration, V1_duration, b1_duration, W2_duration, V2_duration, b2_duration,
           W1_loctime, V1_loctime, b1_loctime, W2_loctime, V2_loctime, b2_loctime,
           L1, bl1, L2, bl2, Ld, bd, Lo, bo):
    WV1 = jnp.stack([
        jnp.concatenate([W1_temporal, V1_temporal], axis=1),
        jnp.concatenate([W1_distance, V1_distance], axis=1),
        jnp.concatenate([W1_duration, V1_duration], axis=1),
        jnp.concatenate([W1_week, V1_week], axis=1),
        jnp.concatenate([W1_weekend, V1_weekend], axis=1),
        jnp.concatenate([W1_loctime, V1_loctime], axis=1),
    ])
    b1a = jnp.concatenate([b1_temporal, b1_distance, b1_duration])[None, :]
    b1s = jnp.stack([b1_week, b1_weekend, b1_loctime])[:, None, :]

    z = jnp.zeros((_H, _C), jnp.float32)
    M2a = jnp.concatenate([
        jnp.concatenate([W2_temporal, z, z, V2_temporal, z, z], axis=1),
        jnp.concatenate([z, W2_distance, z, z, V2_distance, z], axis=1),
        jnp.concatenate([z, z, W2_duration, z, z, V2_duration], axis=1),
    ], axis=0)
    M2s = jnp.stack([
        jnp.concatenate([W2_week, V2_week], axis=1),
        jnp.concatenate([W2_weekend, V2_weekend], axis=1),
        jnp.concatenate([W2_loctime, V2_loctime], axis=1),
    ])
    b2a = jnp.concatenate([b2_temporal, b2_distance, b2_duration])[None, :]
    b2s = jnp.stack([b2_week, b2_weekend, b2_loctime])[:, None, :]

    P = jnp.concatenate([Lo, Lo, Lo, Lo, Lo, 2.0 * Ld, 2.0 * Ld], axis=0)
    bf = (bd + bo)[None, :]

    grid = (_B // _GP,)
    badj = pl.BlockSpec((_GP, _M, _M), lambda b: (b, 0, 0))
    bx = pl.BlockSpec((_GP, _M, _F), lambda b: (b, 0, 0))

    def bcast(shape):
        nd = len(shape)
        return pl.BlockSpec(shape, lambda b: (0,) * nd)

    out = pl.pallas_call(
        _fused_kernel,
        grid=grid,
        in_specs=[badj, badj, badj, badj,
                  bx, bx, bx, bx, bx, bx,
                  bcast((6, _F, 2 * _H)), bcast((1, 3 * _H)),
                  bcast((3, 1, _H)),
                  bcast((3 * _H, 6 * _C)), bcast((3, _H, 2 * _C)),
                  bcast((1, 3 * _C)), bcast((3, 1, _C)),
                  bcast((_F, 40)), bcast((1, 40)),
                  bcast((40, _C)), bcast((1, _C)),
                  bcast((_C * _C, _C * _C)),
                  bcast((_C * _C, _C)), bcast((1, _C))],
        out_specs=pl.BlockSpec((_GP * _M, _C), lambda b: (b, 0)),
        out_shape=jax.ShapeDtypeStruct((_N, _C), jnp.float32),
    )(A_input, A_week_input, A_weekend_input, Location_location_input,
      Temporal_input, Temporal_week_input, Temporal_weekend_input,
      Distance_input, Duration_input, Location_time_input,
      WV1.astype(jnp.bfloat16), b1a, b1s,
      M2a.astype(jnp.bfloat16), M2s.astype(jnp.bfloat16), b2a, b2s,
      L1.astype(jnp.bfloat16), bl1[None, :],
      L2.astype(jnp.bfloat16), bl2[None, :],
      jnp.asarray(_G_BLOCK, dtype=jnp.bfloat16),
      P.astype(jnp.bfloat16), bf)
    return out


# speedup vs baseline: 1.0734x; 1.0023x over previous
import jax
import jax.numpy as jnp
import numpy as np
from jax.experimental import pallas as pl

_B = 64
_M = 128
_F = 48
_C = 7
_H = 20
_N = _B * _M

_GP = 8

_SQRT2 = 1.4142135623730951

_G_BLOCK = np.kron(np.eye(_C, dtype=np.float32),
                   np.ones((_C, _C), dtype=np.float32))


def _gelu(x):
    return 0.5 * x * (1.0 + jax.lax.erf(x / _SQRT2))


def _elu(x):
    return jnp.maximum(x, 0.0) + (jnp.exp(jnp.minimum(x, 0.0)) - 1.0)


def _b16(x):
    return x.astype(jnp.bfloat16)


def _dinv(Ab, ones_col):
    deg = jax.lax.dot_general(Ab, ones_col, (((0,), (0,)), ((), ())),
                              preferred_element_type=jnp.float32)
    safe = jnp.where(deg > 0, deg, 1.0)
    return jnp.where(deg > 0, jax.lax.rsqrt(safe), 0.0)


def _dot(a, b):
    return jnp.dot(_b16(a), b, preferred_element_type=jnp.float32)


def _aggT(An_b, d, h):
    return d * jax.lax.dot_general(
        An_b, _b16(h), (((0,), (0,)), ((), ())),
        preferred_element_type=jnp.float32)


def _fused_kernel(Aa_ref, Aw_ref, Ae_ref, Al_ref,
                  xT_ref, xTw_ref, xTe_ref, xD_ref, xDu_ref, xL_ref,
                  WV1_ref, b1a_ref, b1s_ref, M2a_ref, M2s_ref,
                  b2a_ref, b2s_ref,
                  L1_ref, bl1_ref, L2_ref, bl2_ref,
                  G_ref, P_ref, bf_ref,
                  out_ref):
    G = G_ref[...]
    ones_col = jnp.full((_M, 1), 1.0, dtype=jnp.bfloat16)
    for i in range(_GP):
        Aa = Aa_ref[i]
        Aw = Aw_ref[i]
        Ae = Ae_ref[i]
        Al = Al_ref[i]
        da = _dinv(_b16(Aa), ones_col)
        dw = _dinv(_b16(Aw), ones_col)
        de = _dinv(_b16(Ae), ones_col)
        dl = _dinv(_b16(Al), ones_col)
        Aa = _b16(da * Aa)
        Aw = _b16(dw * Aw)
        Ae = _b16(de * Ae)
        Al = _b16(dl * Al)

        hT = _dot(xT_ref[i], WV1_ref[0])
        hD = _dot(xD_ref[i], WV1_ref[1])
        hDu = _dot(xDu_ref[i], WV1_ref[2])
        hW = _dot(xTw_ref[i], WV1_ref[3])
        hE = _dot(xTe_ref[i], WV1_ref[4])
        hL = _dot(xL_ref[i], WV1_ref[5])

        hw_a = jnp.concatenate([hT[:, :_H], hD[:, :_H], hDu[:, :_H]], axis=1)
        hv_a = jnp.concatenate([hT[:, _H:], hD[:, _H:], hDu[:, _H:]], axis=1)
        h1a = _elu(_gelu(_aggT(Aa, da, hw_a) + hv_a + b1a_ref[...]))
        h1w = _elu(_gelu(_aggT(Aw, dw, hW[:, :_H]) + hW[:, _H:] + b1s_ref[0]))
        h1e = _elu(_gelu(_aggT(Ae, de, hE[:, :_H]) + hE[:, _H:] + b1s_ref[1]))
        h1l = _elu(_gelu(_aggT(Al, dl, hL[:, :_H]) + hL[:, _H:] + b1s_ref[2]))

        H2a = _dot(h1a, M2a_ref[...])
        H2w = _dot(h1w, M2s_ref[0])
        H2e = _dot(h1e, M2s_ref[1])
        H2l = _dot(h1l, M2s_ref[2])
        s_a = jax.nn.relu(_aggT(Aa, da, H2a[:, :3 * _C])
                          + H2a[:, 3 * _C:] + b2a_ref[...])
        s_w = jax.nn.relu(_aggT(Aw, dw, H2w[:, :_C]) + H2w[:, _C:]
                          + b2s_ref[0])
        s_e = jax.nn.relu(_aggT(Ae, de, H2e[:, :_C]) + H2e[:, _C:]
                          + b2s_ref[1])
        s_l = jax.nn.relu(_aggT(Al, dl, H2l[:, :_C]) + H2l[:, _C:]
                          + b2s_ref[2])

        t = jax.nn.relu(_dot(xL_ref[i], L1_ref[...]) + bl1_ref[...])
        s_lt = _dot(t, L2_ref[...]) + bl2_ref[...]

        S = jnp.concatenate([s_a, s_w, s_e, s_l, s_lt], axis=1)
        E = jnp.exp(S - jnp.max(S, axis=-1, keepdims=True))
        En = E / _dot(E, G)
        out_ref[pl.ds(i * _M, _M), :] = _dot(En, P_ref[...]) + bf_ref[...]


def kernel(A_input, A_week_input, A_weekend_input, Location_location_input,
           Temporal_input, Temporal_week_input, Temporal_weekend_input,
           Distance_input, Duration_input, Location_time_input,
           W1_temporal, V1_temporal, b1_temporal, W2_temporal, V2_temporal, b2_temporal,
           W1_week, V1_week, b1_week, W2_week, V2_week, b2_week,
           W1_weekend, V1_weekend, b1_weekend, W2_weekend, V2_weekend, b2_weekend,
           W1_distance, V1_distance, b1_distance, W2_distance, V2_distance, b2_distance,
           W1_duration, V1_duration, b1_duration, W2_duration, V2_duration, b2_duration,
           W1_loctime, V1_loctime, b1_loctime, W2_loctime, V2_loctime, b2_loctime,
           L1, bl1, L2, bl2, Ld, bd, Lo, bo):
    WV1 = jnp.stack([
        jnp.concatenate([W1_temporal, V1_temporal], axis=1),
        jnp.concatenate([W1_distance, V1_distance], axis=1),
        jnp.concatenate([W1_duration, V1_duration], axis=1),
        jnp.concatenate([W1_week, V1_week], axis=1),
        jnp.concatenate([W1_weekend, V1_weekend], axis=1),
        jnp.concatenate([W1_loctime, V1_loctime], axis=1),
    ])
    b1a = jnp.concatenate([b1_temporal, b1_distance, b1_duration])[None, :]
    b1s = jnp.stack([b1_week, b1_weekend, b1_loctime])[:, None, :]

    z = jnp.zeros((_H, _C), jnp.float32)
    M2a = jnp.concatenate([
        jnp.concatenate([W2_temporal, z, z, V2_temporal, z, z], axis=1),
        jnp.concatenate([z, W2_distance, z, z, V2_distance, z], axis=1),
        jnp.concatenate([z, z, W2_duration, z, z, V2_duration], axis=1),
    ], axis=0)
    M2s = jnp.stack([
        jnp.concatenate([W2_week, V2_week], axis=1),
        jnp.concatenate([W2_weekend, V2_weekend], axis=1),
        jnp.concatenate([W2_loctime, V2_loctime], axis=1),
    ])
    b2a = jnp.concatenate([b2_temporal, b2_distance, b2_duration])[None, :]
    b2s = jnp.stack([b2_week, b2_weekend, b2_loctime])[:, None, :]

    P = jnp.concatenate([Lo, Lo, Lo, Lo, Lo, 2.0 * Ld, 2.0 * Ld], axis=0)
    bf = (bd + bo)[None, :]

    grid = (_B // _GP,)
    badj = pl.BlockSpec((_GP, _M, _M), lambda b: (b, 0, 0))
    bx = pl.BlockSpec((_GP, _M, _F), lambda b: (b, 0, 0))

    def bcast(shape):
        nd = len(shape)
        return pl.BlockSpec(shape, lambda b: (0,) * nd)

    out = pl.pallas_call(
        _fused_kernel,
        grid=grid,
        in_specs=[badj, badj, badj, badj,
                  bx, bx, bx, bx, bx, bx,
                  bcast((6, _F, 2 * _H)), bcast((1, 3 * _H)),
                  bcast((3, 1, _H)),
                  bcast((3 * _H, 6 * _C)), bcast((3, _H, 2 * _C)),
                  bcast((1, 3 * _C)), bcast((3, 1, _C)),
                  bcast((_F, 40)), bcast((1, 40)),
                  bcast((40, _C)), bcast((1, _C)),
                  bcast((_C * _C, _C * _C)),
                  bcast((_C * _C, _C)), bcast((1, _C))],
        out_specs=pl.BlockSpec((_GP * _M, _C), lambda b: (b, 0)),
        out_shape=jax.ShapeDtypeStruct((_N, _C), jnp.float32),
    )(A_input, A_week_input, A_weekend_input, Location_location_input,
      Temporal_input, Temporal_week_input, Temporal_weekend_input,
      Distance_input, Duration_input, Location_time_input,
      WV1.astype(jnp.bfloat16), b1a, b1s,
      M2a.astype(jnp.bfloat16), M2s.astype(jnp.bfloat16), b2a, b2s,
      L1.astype(jnp.bfloat16), bl1[None, :],
      L2.astype(jnp.bfloat16), bl2[None, :],
      jnp.asarray(_G_BLOCK, dtype=jnp.bfloat16),
      P.astype(jnp.bfloat16), bf)
    return out


# speedup vs baseline: 1.0776x; 1.0039x over previous
import jax
import jax.numpy as jnp
import numpy as np
from jax.experimental import pallas as pl
from jax.experimental.pallas import tpu as pltpu

_B = 64
_M = 128
_F = 48
_C = 7
_H = 20
_N = _B * _M

_GP = 8

_SQRT2 = 1.4142135623730951

_G_BLOCK = np.kron(np.eye(_C, dtype=np.float32),
                   np.ones((_C, _C), dtype=np.float32))


def _gelu(x):
    return 0.5 * x * (1.0 + jax.lax.erf(x / _SQRT2))


def _elu(x):
    return jnp.maximum(x, 0.0) + (jnp.exp(jnp.minimum(x, 0.0)) - 1.0)


def _b16(x):
    return x.astype(jnp.bfloat16)


def _dinv(Ab, ones_col):
    deg = jax.lax.dot_general(Ab, ones_col, (((0,), (0,)), ((), ())),
                              preferred_element_type=jnp.float32)
    safe = jnp.where(deg > 0, deg, 1.0)
    return jnp.where(deg > 0, jax.lax.rsqrt(safe), 0.0)


def _dot(a, b):
    return jnp.dot(_b16(a), b, preferred_element_type=jnp.float32)


def _aggT(An_b, d, h):
    return d * jax.lax.dot_general(
        An_b, _b16(h), (((0,), (0,)), ((), ())),
        preferred_element_type=jnp.float32)


def _fused_kernel(Aa_ref, Aw_ref, Ae_ref, Al_ref,
                  xT_ref, xTw_ref, xTe_ref, xD_ref, xDu_ref, xL_ref,
                  WV1_ref, b1a_ref, b1s_ref, M2a_ref, M2s_ref,
                  b2a_ref, b2s_ref,
                  L1_ref, bl1_ref, L2_ref, bl2_ref,
                  G_ref, P_ref, bf_ref,
                  out_ref):
    G = G_ref[...]
    ones_col = jnp.full((_M, 1), 1.0, dtype=jnp.bfloat16)
    for i in range(_GP):
        Aa = Aa_ref[i]
        Aw = Aw_ref[i]
        Ae = Ae_ref[i]
        Al = Al_ref[i]
        da = _dinv(_b16(Aa), ones_col)
        dw = _dinv(_b16(Aw), ones_col)
        de = _dinv(_b16(Ae), ones_col)
        dl = _dinv(_b16(Al), ones_col)
        Aa = _b16(da * Aa)
        Aw = _b16(dw * Aw)
        Ae = _b16(de * Ae)
        Al = _b16(dl * Al)

        hT = _dot(xT_ref[i], WV1_ref[0])
        hD = _dot(xD_ref[i], WV1_ref[1])
        hDu = _dot(xDu_ref[i], WV1_ref[2])
        hW = _dot(xTw_ref[i], WV1_ref[3])
        hE = _dot(xTe_ref[i], WV1_ref[4])
        hL = _dot(xL_ref[i], WV1_ref[5])

        hw_a = jnp.concatenate([hT[:, :_H], hD[:, :_H], hDu[:, :_H]], axis=1)
        hv_a = jnp.concatenate([hT[:, _H:], hD[:, _H:], hDu[:, _H:]], axis=1)
        h1a = _elu(_gelu(_aggT(Aa, da, hw_a) + hv_a + b1a_ref[...]))
        h1w = _elu(_gelu(_aggT(Aw, dw, hW[:, :_H]) + hW[:, _H:] + b1s_ref[0]))
        h1e = _elu(_gelu(_aggT(Ae, de, hE[:, :_H]) + hE[:, _H:] + b1s_ref[1]))
        h1l = _elu(_gelu(_aggT(Al, dl, hL[:, :_H]) + hL[:, _H:] + b1s_ref[2]))

        H2a = _dot(h1a, M2a_ref[...])
        H2w = _dot(h1w, M2s_ref[0])
        H2e = _dot(h1e, M2s_ref[1])
        H2l = _dot(h1l, M2s_ref[2])
        s_a = jax.nn.relu(_aggT(Aa, da, H2a[:, :3 * _C])
                          + H2a[:, 3 * _C:] + b2a_ref[...])
        s_w = jax.nn.relu(_aggT(Aw, dw, H2w[:, :_C]) + H2w[:, _C:]
                          + b2s_ref[0])
        s_e = jax.nn.relu(_aggT(Ae, de, H2e[:, :_C]) + H2e[:, _C:]
                          + b2s_ref[1])
        s_l = jax.nn.relu(_aggT(Al, dl, H2l[:, :_C]) + H2l[:, _C:]
                          + b2s_ref[2])

        t = jax.nn.relu(_dot(xL_ref[i], L1_ref[...]) + bl1_ref[...])
        s_lt = _dot(t, L2_ref[...]) + bl2_ref[...]

        S = jnp.concatenate([s_a, s_w, s_e, s_l, s_lt], axis=1)
        E = jnp.exp(S - jnp.max(S, axis=-1, keepdims=True))
        En = E / _dot(E, G)
        out_ref[pl.ds(i * _M, _M), :] = _dot(En, P_ref[...]) + bf_ref[...]


def kernel(A_input, A_week_input, A_weekend_input, Location_location_input,
           Temporal_input, Temporal_week_input, Temporal_weekend_input,
           Distance_input, Duration_input, Location_time_input,
           W1_temporal, V1_temporal, b1_temporal, W2_temporal, V2_temporal, b2_temporal,
           W1_week, V1_week, b1_week, W2_week, V2_week, b2_week,
           W1_weekend, V1_weekend, b1_weekend, W2_weekend, V2_weekend, b2_weekend,
           W1_distance, V1_distance, b1_distance, W2_distance, V2_distance, b2_distance,
           W1_duration, V1_duration, b1_duration, W2_duration, V2_duration, b2_duration,
           W1_loctime, V1_loctime, b1_loctime, W2_loctime, V2_loctime, b2_loctime,
           L1, bl1, L2, bl2, Ld, bd, Lo, bo):
    WV1 = jnp.stack([
        jnp.concatenate([W1_temporal, V1_temporal], axis=1),
        jnp.concatenate([W1_distance, V1_distance], axis=1),
        jnp.concatenate([W1_duration, V1_duration], axis=1),
        jnp.concatenate([W1_week, V1_week], axis=1),
        jnp.concatenate([W1_weekend, V1_weekend], axis=1),
        jnp.concatenate([W1_loctime, V1_loctime], axis=1),
    ])
    b1a = jnp.concatenate([b1_temporal, b1_distance, b1_duration])[None, :]
    b1s = jnp.stack([b1_week, b1_weekend, b1_loctime])[:, None, :]

    z = jnp.zeros((_H, _C), jnp.float32)
    M2a = jnp.concatenate([
        jnp.concatenate([W2_temporal, z, z, V2_temporal, z, z], axis=1),
        jnp.concatenate([z, W2_distance, z, z, V2_distance, z], axis=1),
        jnp.concatenate([z, z, W2_duration, z, z, V2_duration], axis=1),
    ], axis=0)
    M2s = jnp.stack([
        jnp.concatenate([W2_week, V2_week], axis=1),
        jnp.concatenate([W2_weekend, V2_weekend], axis=1),
        jnp.concatenate([W2_loctime, V2_loctime], axis=1),
    ])
    b2a = jnp.concatenate([b2_temporal, b2_distance, b2_duration])[None, :]
    b2s = jnp.stack([b2_week, b2_weekend, b2_loctime])[:, None, :]

    P = jnp.concatenate([Lo, Lo, Lo, Lo, Lo, 2.0 * Ld, 2.0 * Ld], axis=0)
    bf = (bd + bo)[None, :]

    grid = (_B // _GP,)
    badj = pl.BlockSpec((_GP, _M, _M), lambda b: (b, 0, 0))
    bx = pl.BlockSpec((_GP, _M, _F), lambda b: (b, 0, 0))

    def bcast(shape):
        nd = len(shape)
        return pl.BlockSpec(shape, lambda b: (0,) * nd)

    out = pl.pallas_call(
        _fused_kernel,
        grid=grid,
        in_specs=[badj, badj, badj, badj,
                  bx, bx, bx, bx, bx, bx,
                  bcast((6, _F, 2 * _H)), bcast((1, 3 * _H)),
                  bcast((3, 1, _H)),
                  bcast((3 * _H, 6 * _C)), bcast((3, _H, 2 * _C)),
                  bcast((1, 3 * _C)), bcast((3, 1, _C)),
                  bcast((_F, 40)), bcast((1, 40)),
                  bcast((40, _C)), bcast((1, _C)),
                  bcast((_C * _C, _C * _C)),
                  bcast((_C * _C, _C)), bcast((1, _C))],
        out_specs=pl.BlockSpec((_GP * _M, _C), lambda b: (b, 0)),
        out_shape=jax.ShapeDtypeStruct((_N, _C), jnp.float32),
        compiler_params=pltpu.CompilerParams(
            dimension_semantics=("parallel",)),
    )(A_input, A_week_input, A_weekend_input, Location_location_input,
      Temporal_input, Temporal_week_input, Temporal_weekend_input,
      Distance_input, Duration_input, Location_time_input,
      WV1.astype(jnp.bfloat16), b1a, b1s,
      M2a.astype(jnp.bfloat16), M2s.astype(jnp.bfloat16), b2a, b2s,
      L1.astype(jnp.bfloat16), bl1[None, :],
      L2.astype(jnp.bfloat16), bl2[None, :],
      jnp.asarray(_G_BLOCK, dtype=jnp.bfloat16),
      P.astype(jnp.bfloat16), bf)
    return out
